# baseline (speedup 1.0000x reference)
import functools

import jax
import jax.numpy as jnp
from jax import lax
from jax.experimental import pallas as pl
from jax.experimental.pallas import tpu as pltpu
from jax.experimental.pallas import tpu_sc as plsc

_N = 10000
_E = 320000
_F_IN = 128
_HID = 64
_C = 16

_NC = 2
_NS = 16
_NW = _NC * _NS
_CHUNK = 125
_ROWS = _E // _CHUNK
_ROWS_W = _ROWS // _NW
_NPAD = 10240
_NPS = _NPAD // _NS

_ECHUNK = 500
_EROWS = _E // _ECHUNK
_EROWS_W = _EROWS // _NW

_mesh = plsc.VectorSubcoreMesh(
    core_axis_name="c", subcore_axis_name="s", num_cores=_NC, num_subcores=_NS
)


@functools.partial(
    pl.kernel,
    out_type=[
        jax.ShapeDtypeStruct((_NPAD,), jnp.float32),
        jax.ShapeDtypeStruct((_NPAD,), jnp.float32),
    ],
    mesh=_mesh,
    compiler_params=pltpu.CompilerParams(use_tc_tiling_on_sc=False),
    scratch_types=[
        pltpu.VMEM((_EROWS_W, _ECHUNK), jnp.int32),
        pltpu.VMEM((512,), jnp.float32),
        pltpu.VMEM((_NPS,), jnp.float32),
        pltpu.VMEM_SHARED((_NPAD,), jnp.float32),
    ],
)
def _sc_degree(edge_hbm, cnt0_hbm, cnt1_hbm, colv, ones_v, zbuf, acc):
    cid = lax.axis_index("c")
    sid = lax.axis_index("s")
    wid = sid * _NC + cid
    for k in range(32):
        ones_v[pl.ds(k * 16, 16)] = jnp.ones((16,), jnp.float32)

    def zfill(k, carry):
        zbuf[pl.ds(k * 16, 16)] = jnp.zeros((16,), jnp.float32)
        return carry

    lax.fori_loop(0, _NPS // 16, zfill, 0)
    pltpu.sync_copy(zbuf, acc.at[pl.ds(sid * _NPS, _NPS)])
    pltpu.sync_copy(edge_hbm.at[1, pl.ds(wid * _EROWS_W, _EROWS_W)], colv)
    plsc.subcore_barrier()

    def body(j, carry):
        pltpu.sync_copy(ones_v.at[pl.ds(0, _ECHUNK)], acc.at[colv.at[j]], add=True)
        return carry

    lax.fori_loop(0, _EROWS_W, body, 0)
    plsc.subcore_barrier()

    @pl.when(cid == 0)
    def _():
        pltpu.sync_copy(acc.at[pl.ds(sid * _NPS, _NPS)], cnt0_hbm.at[pl.ds(sid * _NPS, _NPS)])

    @pl.when(cid == 1)
    def _():
        pltpu.sync_copy(acc.at[pl.ds(sid * _NPS, _NPS)], cnt1_hbm.at[pl.ds(sid * _NPS, _NPS)])


def _make_sc_scatter(depth, nbuf, concat_out):
    if concat_out:
        out_type = [jax.ShapeDtypeStruct((_NPAD, 2 * depth), jnp.float32)]
    else:
        out_type = [
            jax.ShapeDtypeStruct((_NPAD, depth), jnp.float32),
            jax.ShapeDtypeStruct((_NPAD, depth), jnp.float32),
        ]

    @functools.partial(
        pl.kernel,
        out_type=out_type,
        mesh=_mesh,
        compiler_params=pltpu.CompilerParams(use_tc_tiling_on_sc=False),
        scratch_types=(
            [
                pltpu.VMEM((_EROWS_W, _ECHUNK), jnp.int32),
                pltpu.VMEM((_EROWS_W, _ECHUNK), jnp.int32),
            ]
            + [pltpu.VMEM((_ECHUNK, depth), jnp.float32)] * nbuf
            + [pltpu.VMEM_SHARED((_NPAD, depth), jnp.float32)]
            + [pltpu.SemaphoreType.DMA] * nbuf
        ),
    )
    def _sc_scatter(edge_hbm, y_hbm, zd_hbm, *rest):
        if concat_out:
            p01_hbm = rest[0]
            rest = rest[1:]
        else:
            p0_hbm, p1_hbm = rest[:2]
            rest = rest[2:]
        rowv, colv = rest[:2]
        bufs = rest[2:2 + nbuf]
        acc = rest[2 + nbuf]
        sems = rest[3 + nbuf:]
        cid = lax.axis_index("c")
        sid = lax.axis_index("s")
        wid = sid * _NC + cid

        pltpu.sync_copy(zd_hbm.at[pl.ds(sid * _NPS, _NPS)],
                        acc.at[pl.ds(sid * _NPS, _NPS)])
        pltpu.sync_copy(edge_hbm.at[0, pl.ds(wid * _EROWS_W, _EROWS_W)], rowv)
        pltpu.sync_copy(edge_hbm.at[1, pl.ds(wid * _EROWS_W, _EROWS_W)], colv)
        plsc.subcore_barrier()

        for b in range(nbuf):
            pltpu.async_copy(y_hbm.at[rowv.at[b]], bufs[b], sems[b])

        def body(i, carry):
            for b in range(nbuf):
                j = nbuf * i + b
                pltpu.make_async_copy(y_hbm.at[rowv.at[j]], bufs[b], sems[b]).wait()
                pltpu.sync_copy(bufs[b], acc.at[colv.at[j]], add=True)

                @pl.when(j + nbuf < _EROWS_W)
                def _():
                    pltpu.async_copy(y_hbm.at[rowv.at[j + nbuf]], bufs[b], sems[b])

            return carry

        lax.fori_loop(0, _EROWS_W // nbuf, body, 0)
        plsc.subcore_barrier()

        if concat_out:
            pltpu.sync_copy(
                acc.at[pl.ds(sid * _NPS, _NPS)],
                p01_hbm.at[pl.ds(sid * _NPS, _NPS), pl.ds(cid * depth, depth)])
        else:
            @pl.when(cid == 0)
            def _():
                pltpu.sync_copy(acc.at[pl.ds(sid * _NPS, _NPS)],
                                p0_hbm.at[pl.ds(sid * _NPS, _NPS)])

            @pl.when(cid == 1)
            def _():
                pltpu.sync_copy(acc.at[pl.ds(sid * _NPS, _NPS)],
                                p1_hbm.at[pl.ds(sid * _NPS, _NPS)])

    return _sc_scatter


_sc_scatter_hid = _make_sc_scatter(_HID, 2, True)
_sc_scatter_out = _make_sc_scatter(_C, 4, True)


_R = 1000
_G = _N // _R


def _tc1a_body(x_ref, w1_ref, w2_ref, xw_ref, o_ref):
    xw_ref[...] = jnp.dot(x_ref[...], w1_ref[...], preferred_element_type=jnp.float32)

    @pl.when(pl.program_id(0) == 0)
    def _():
        w1 = w1_ref[...]
        w2 = w2_ref[...]
        g1 = lax.dot_general(w1, w1, (((1,), (1,)), ((), ())),
                             preferred_element_type=jnp.float32)
        g2 = lax.dot_general(w2, w2, (((1,), (1,)), ((), ())),
                             preferred_element_type=jnp.float32)
        i1 = (lax.broadcasted_iota(jnp.int32, (_F_IN, _F_IN), 0)
              == lax.broadcasted_iota(jnp.int32, (_F_IN, _F_IN), 1)).astype(jnp.float32)
        i2 = (lax.broadcasted_iota(jnp.int32, (_HID, _HID), 0)
              == lax.broadcasted_iota(jnp.int32, (_HID, _HID), 1)).astype(jnp.float32)
        s1 = jnp.sum((g1 - i1) ** 2)
        s2 = jnp.sum((g2 - i2) ** 2)
        o_ref[...] = jnp.reshape(jnp.sqrt(s1) + jnp.sqrt(s2), (1, 1))


_tc1a = pl.pallas_call(
    _tc1a_body,
    grid=(_G,),
    in_specs=[
        pl.BlockSpec((_R, _F_IN), lambda i: (i, 0)),
        pl.BlockSpec((_F_IN, _HID), lambda i: (0, 0)),
        pl.BlockSpec((_HID, _C), lambda i: (0, 0)),
    ],
    out_specs=[
        pl.BlockSpec((_R, _HID), lambda i: (i, 0)),
        pl.BlockSpec((1, 1), lambda i: (0, 0)),
    ],
    out_shape=[
        jax.ShapeDtypeStruct((_N, _HID), jnp.float32),
        jax.ShapeDtypeStruct((1, 1), jnp.float32),
    ],
)


def _tc1b_body(xw_ref, c0_ref, c1_ref, y_ref, dinv_ref):
    deg = c0_ref[...] + c1_ref[...] + 1.0
    dinv = lax.rsqrt(deg)
    y_ref[...] = xw_ref[...] * dinv
    dinv_ref[...] = dinv


_tc1b = pl.pallas_call(
    _tc1b_body,
    grid=(_G,),
    in_specs=[
        pl.BlockSpec((_R, _HID), lambda i: (i, 0)),
        pl.BlockSpec((_R, 1), lambda i: (i, 0)),
        pl.BlockSpec((_R, 1), lambda i: (i, 0)),
    ],
    out_specs=[
        pl.BlockSpec((_R, _HID), lambda i: (i, 0)),
        pl.BlockSpec((_R, 1), lambda i: (i, 0)),
    ],
    out_shape=[
        jax.ShapeDtypeStruct((_N, _HID), jnp.float32),
        jax.ShapeDtypeStruct((_N, 1), jnp.float32),
    ],
)


def _tc2_body(p01_ref, y1_ref, dinv_ref, b1_ref, w2_ref, z_ref):
    dinv = dinv_ref[...]
    p01 = p01_ref[...]
    out1 = (p01[:, :_HID] + p01[:, _HID:] + y1_ref[...]) * dinv + b1_ref[...]
    h = jnp.maximum(out1, 0.0)
    z_ref[...] = jnp.dot(h, w2_ref[...], preferred_element_type=jnp.float32) * dinv


_tc2 = pl.pallas_call(
    _tc2_body,
    grid=(_G,),
    in_specs=[
        pl.BlockSpec((_R, 2 * _HID), lambda i: (i, 0)),
        pl.BlockSpec((_R, _HID), lambda i: (i, 0)),
        pl.BlockSpec((_R, 1), lambda i: (i, 0)),
        pl.BlockSpec((1, _HID), lambda i: (0, 0)),
        pl.BlockSpec((_HID, _C), lambda i: (0, 0)),
    ],
    out_specs=[pl.BlockSpec((_R, _C), lambda i: (i, 0))],
    out_shape=[jax.ShapeDtypeStruct((_N, _C), jnp.float32)],
)


def _tc3_body(q01_ref, z2_ref, dinv_ref, b2_ref, logp_ref, xout_ref):
    q01 = q01_ref[...]
    xo = (q01[:, :_C] + q01[:, _C:] + z2_ref[...]) * dinv_ref[...] + b2_ref[...]
    m = jnp.max(xo, axis=1, keepdims=True)
    t = xo - m
    lse = jnp.log(jnp.sum(jnp.exp(t), axis=1, keepdims=True))
    logp_ref[...] = t - lse
    xout_ref[...] = xo


_tc3 = pl.pallas_call(
    _tc3_body,
    grid=(_G,),
    in_specs=[
        pl.BlockSpec((_R, 2 * _C), lambda i: (i, 0)),
        pl.BlockSpec((_R, _C), lambda i: (i, 0)),
        pl.BlockSpec((_R, 1), lambda i: (i, 0)),
        pl.BlockSpec((1, _C), lambda i: (0, 0)),
    ],
    out_specs=[
        pl.BlockSpec((_R, _C), lambda i: (i, 0)),
        pl.BlockSpec((_R, _C), lambda i: (i, 0)),
    ],
    out_shape=[
        jax.ShapeDtypeStruct((_N, _C), jnp.float32),
        jax.ShapeDtypeStruct((_N, _C), jnp.float32),
    ],
)


def kernel(x, edge_index, W1, b1, W2, b2):
    edge_r = edge_index.reshape(2, _EROWS, _ECHUNK)
    z64 = jnp.zeros((_NPAD, _HID), jnp.float32)
    z16 = jnp.zeros((_NPAD, _C), jnp.float32)

    cnt0, cnt1 = _sc_degree(edge_r)
    xw1, orto = _tc1a(x, W1, W2)
    y1, dinv = _tc1b(xw1, cnt0.reshape(_NPAD, 1)[: _N], cnt1.reshape(_NPAD, 1)[: _N])
    (p01,) = _sc_scatter_hid(edge_r, y1, z64)
    (z2,) = _tc2(p01, y1, dinv, b1.reshape(1, _HID), W2)
    (q01,) = _sc_scatter_out(edge_r, z2, z16)
    logp, xout = _tc3(q01, z2, dinv, b2.reshape(1, _C))
    return (logp, xout, orto.reshape(()))


# speedup vs baseline: 1.0156x; 1.0156x over previous
import functools

import jax
import jax.numpy as jnp
from jax import lax
from jax.experimental import pallas as pl
from jax.experimental.pallas import tpu as pltpu
from jax.experimental.pallas import tpu_sc as plsc

_N = 10000
_E = 320000
_F_IN = 128
_HID = 64
_C = 16

_NC = 2
_NS = 16
_NW = _NC * _NS
_CHUNK = 125
_ROWS = _E // _CHUNK
_ROWS_W = _ROWS // _NW
_NPAD = 10240
_NPS = _NPAD // _NS

_ECHUNK = 500
_EROWS = _E // _ECHUNK
_EROWS_W = _EROWS // _NW

_mesh = plsc.VectorSubcoreMesh(
    core_axis_name="c", subcore_axis_name="s", num_cores=_NC, num_subcores=_NS
)


@functools.partial(
    pl.kernel,
    out_type=[
        jax.ShapeDtypeStruct((_NPAD,), jnp.float32),
        jax.ShapeDtypeStruct((_NPAD,), jnp.float32),
    ],
    mesh=_mesh,
    compiler_params=pltpu.CompilerParams(use_tc_tiling_on_sc=False),
    scratch_types=[
        pltpu.VMEM((_EROWS_W, _ECHUNK), jnp.int32),
        pltpu.VMEM((512,), jnp.float32),
        pltpu.VMEM((_NPS,), jnp.float32),
        pltpu.VMEM_SHARED((_NPAD,), jnp.float32),
    ],
)
def _sc_degree(edge_hbm, cnt0_hbm, cnt1_hbm, colv, ones_v, zbuf, acc):
    cid = lax.axis_index("c")
    sid = lax.axis_index("s")
    wid = sid * _NC + cid
    for k in range(32):
        ones_v[pl.ds(k * 16, 16)] = jnp.ones((16,), jnp.float32)

    def zfill(k, carry):
        zbuf[pl.ds(k * 16, 16)] = jnp.zeros((16,), jnp.float32)
        return carry

    lax.fori_loop(0, _NPS // 16, zfill, 0)
    pltpu.sync_copy(zbuf, acc.at[pl.ds(sid * _NPS, _NPS)])
    pltpu.sync_copy(edge_hbm.at[1, pl.ds(wid * _EROWS_W, _EROWS_W)], colv)
    plsc.subcore_barrier()

    def body(j, carry):
        pltpu.sync_copy(ones_v.at[pl.ds(0, _ECHUNK)], acc.at[colv.at[j]], add=True)
        return carry

    lax.fori_loop(0, _EROWS_W, body, 0)
    plsc.subcore_barrier()

    @pl.when(cid == 0)
    def _():
        pltpu.sync_copy(acc.at[pl.ds(sid * _NPS, _NPS)], cnt0_hbm.at[pl.ds(sid * _NPS, _NPS)])

    @pl.when(cid == 1)
    def _():
        pltpu.sync_copy(acc.at[pl.ds(sid * _NPS, _NPS)], cnt1_hbm.at[pl.ds(sid * _NPS, _NPS)])


def _make_sc_scatter(depth, nbuf, concat_out):
    if concat_out:
        out_type = [jax.ShapeDtypeStruct((_NPAD, 2 * depth), jnp.float32)]
    else:
        out_type = [
            jax.ShapeDtypeStruct((_NPAD, depth), jnp.float32),
            jax.ShapeDtypeStruct((_NPAD, depth), jnp.float32),
        ]

    @functools.partial(
        pl.kernel,
        out_type=out_type,
        mesh=_mesh,
        compiler_params=pltpu.CompilerParams(use_tc_tiling_on_sc=False),
        scratch_types=(
            [
                pltpu.VMEM((_EROWS_W, _ECHUNK), jnp.int32),
                pltpu.VMEM((_EROWS_W, _ECHUNK), jnp.int32),
            ]
            + [pltpu.VMEM((_ECHUNK, depth), jnp.float32)] * nbuf
            + [pltpu.VMEM_SHARED((_NPAD, depth), jnp.float32)]
            + [pltpu.SemaphoreType.DMA] * nbuf
        ),
    )
    def _sc_scatter(edge_hbm, y_hbm, zd_hbm, *rest):
        if concat_out:
            p01_hbm = rest[0]
            rest = rest[1:]
        else:
            p0_hbm, p1_hbm = rest[:2]
            rest = rest[2:]
        rowv, colv = rest[:2]
        bufs = rest[2:2 + nbuf]
        acc = rest[2 + nbuf]
        sems = rest[3 + nbuf:]
        cid = lax.axis_index("c")
        sid = lax.axis_index("s")
        wid = sid * _NC + cid

        pltpu.sync_copy(zd_hbm.at[pl.ds(sid * _NPS, _NPS)],
                        acc.at[pl.ds(sid * _NPS, _NPS)])
        pltpu.sync_copy(edge_hbm.at[0, pl.ds(wid * _EROWS_W, _EROWS_W)], rowv)
        pltpu.sync_copy(edge_hbm.at[1, pl.ds(wid * _EROWS_W, _EROWS_W)], colv)
        plsc.subcore_barrier()

        for b in range(nbuf):
            pltpu.async_copy(y_hbm.at[rowv.at[b]], bufs[b], sems[b])

        def body(i, carry):
            for b in range(nbuf):
                j = nbuf * i + b
                pltpu.make_async_copy(y_hbm.at[rowv.at[j]], bufs[b], sems[b]).wait()
                pltpu.sync_copy(bufs[b], acc.at[colv.at[j]], add=True)

                @pl.when(j + nbuf < _EROWS_W)
                def _():
                    pltpu.async_copy(y_hbm.at[rowv.at[j + nbuf]], bufs[b], sems[b])

            return carry

        lax.fori_loop(0, _EROWS_W // nbuf, body, 0)
        plsc.subcore_barrier()

        if concat_out:
            pltpu.sync_copy(
                acc.at[pl.ds(sid * _NPS, _NPS)],
                p01_hbm.at[pl.ds(sid * _NPS, _NPS), pl.ds(cid * depth, depth)])
        else:
            @pl.when(cid == 0)
            def _():
                pltpu.sync_copy(acc.at[pl.ds(sid * _NPS, _NPS)],
                                p0_hbm.at[pl.ds(sid * _NPS, _NPS)])

            @pl.when(cid == 1)
            def _():
                pltpu.sync_copy(acc.at[pl.ds(sid * _NPS, _NPS)],
                                p1_hbm.at[pl.ds(sid * _NPS, _NPS)])

    return _sc_scatter


_sc_scatter_hid = _make_sc_scatter(_HID, 2, True)
_sc_scatter_out = _make_sc_scatter(_C, 4, True)


_R = 1000
_G = _N // _R


def _tc1a_body(x_ref, w1_ref, w2_ref, c0_ref, c1_ref, y_ref, dinv_ref, o_ref):
    xw = jnp.dot(x_ref[...], w1_ref[...], preferred_element_type=jnp.float32)
    deg = c0_ref[...] + c1_ref[...] + 1.0
    dinv = lax.rsqrt(deg)
    y_ref[...] = xw * dinv
    dinv_ref[...] = dinv

    @pl.when(pl.program_id(0) == 0)
    def _():
        w1 = w1_ref[...]
        w2 = w2_ref[...]
        g1 = lax.dot_general(w1, w1, (((1,), (1,)), ((), ())),
                             preferred_element_type=jnp.float32)
        g2 = lax.dot_general(w2, w2, (((1,), (1,)), ((), ())),
                             preferred_element_type=jnp.float32)
        i1 = (lax.broadcasted_iota(jnp.int32, (_F_IN, _F_IN), 0)
              == lax.broadcasted_iota(jnp.int32, (_F_IN, _F_IN), 1)).astype(jnp.float32)
        i2 = (lax.broadcasted_iota(jnp.int32, (_HID, _HID), 0)
              == lax.broadcasted_iota(jnp.int32, (_HID, _HID), 1)).astype(jnp.float32)
        s1 = jnp.sum((g1 - i1) ** 2)
        s2 = jnp.sum((g2 - i2) ** 2)
        o_ref[...] = jnp.reshape(jnp.sqrt(s1) + jnp.sqrt(s2), (1, 1))


_tc1a = pl.pallas_call(
    _tc1a_body,
    grid=(_G,),
    in_specs=[
        pl.BlockSpec((_R, _F_IN), lambda i: (i, 0)),
        pl.BlockSpec((_F_IN, _HID), lambda i: (0, 0)),
        pl.BlockSpec((_HID, _C), lambda i: (0, 0)),
        pl.BlockSpec((_R, 1), lambda i: (i, 0)),
        pl.BlockSpec((_R, 1), lambda i: (i, 0)),
    ],
    out_specs=[
        pl.BlockSpec((_R, _HID), lambda i: (i, 0)),
        pl.BlockSpec((_R, 1), lambda i: (i, 0)),
        pl.BlockSpec((1, 1), lambda i: (0, 0)),
    ],
    out_shape=[
        jax.ShapeDtypeStruct((_N, _HID), jnp.float32),
        jax.ShapeDtypeStruct((_N, 1), jnp.float32),
        jax.ShapeDtypeStruct((1, 1), jnp.float32),
    ],
)


def _tc2_body(p01_ref, y1_ref, dinv_ref, b1_ref, w2_ref, z_ref):
    dinv = dinv_ref[...]
    p01 = p01_ref[...]
    out1 = (p01[:, :_HID] + p01[:, _HID:] + y1_ref[...]) * dinv + b1_ref[...]
    h = jnp.maximum(out1, 0.0)
    z_ref[...] = jnp.dot(h, w2_ref[...], preferred_element_type=jnp.float32) * dinv


_tc2 = pl.pallas_call(
    _tc2_body,
    grid=(_G,),
    in_specs=[
        pl.BlockSpec((_R, 2 * _HID), lambda i: (i, 0)),
        pl.BlockSpec((_R, _HID), lambda i: (i, 0)),
        pl.BlockSpec((_R, 1), lambda i: (i, 0)),
        pl.BlockSpec((1, _HID), lambda i: (0, 0)),
        pl.BlockSpec((_HID, _C), lambda i: (0, 0)),
    ],
    out_specs=[pl.BlockSpec((_R, _C), lambda i: (i, 0))],
    out_shape=[jax.ShapeDtypeStruct((_N, _C), jnp.float32)],
)


def _tc3_body(q01_ref, z2_ref, dinv_ref, b2_ref, logp_ref, xout_ref):
    q01 = q01_ref[...]
    xo = (q01[:, :_C] + q01[:, _C:] + z2_ref[...]) * dinv_ref[...] + b2_ref[...]
    m = jnp.max(xo, axis=1, keepdims=True)
    t = xo - m
    lse = jnp.log(jnp.sum(jnp.exp(t), axis=1, keepdims=True))
    logp_ref[...] = t - lse
    xout_ref[...] = xo


_tc3 = pl.pallas_call(
    _tc3_body,
    grid=(_G,),
    in_specs=[
        pl.BlockSpec((_R, 2 * _C), lambda i: (i, 0)),
        pl.BlockSpec((_R, _C), lambda i: (i, 0)),
        pl.BlockSpec((_R, 1), lambda i: (i, 0)),
        pl.BlockSpec((1, _C), lambda i: (0, 0)),
    ],
    out_specs=[
        pl.BlockSpec((_R, _C), lambda i: (i, 0)),
        pl.BlockSpec((_R, _C), lambda i: (i, 0)),
    ],
    out_shape=[
        jax.ShapeDtypeStruct((_N, _C), jnp.float32),
        jax.ShapeDtypeStruct((_N, _C), jnp.float32),
    ],
)


def kernel(x, edge_index, W1, b1, W2, b2):
    edge_r = edge_index.reshape(2, _EROWS, _ECHUNK)
    z64 = jnp.zeros((_NPAD, _HID), jnp.float32)
    z16 = jnp.zeros((_NPAD, _C), jnp.float32)

    cnt0, cnt1 = _sc_degree(edge_r)
    y1, dinv, orto = _tc1a(x, W1, W2, cnt0.reshape(_NPAD, 1)[: _N],
                           cnt1.reshape(_NPAD, 1)[: _N])
    (p01,) = _sc_scatter_hid(edge_r, y1, z64)
    (z2,) = _tc2(p01, y1, dinv, b1.reshape(1, _HID), W2)
    (q01,) = _sc_scatter_out(edge_r, z2, z16)
    logp, xout = _tc3(q01, z2, dinv, b2.reshape(1, _C))
    return (logp, xout, orto.reshape(()))


# speedup vs baseline: 1.0500x; 1.0339x over previous
import functools

import jax
import jax.numpy as jnp
from jax import lax
from jax.experimental import pallas as pl
from jax.experimental.pallas import tpu as pltpu
from jax.experimental.pallas import tpu_sc as plsc

_N = 10000
_E = 320000
_F_IN = 128
_HID = 64
_C = 16

_NC = 2
_NS = 16
_NW = _NC * _NS
_CHUNK = 125
_ROWS = _E // _CHUNK
_ROWS_W = _ROWS // _NW
_NPAD = 10240
_NPS = _NPAD // _NS

_ECHUNK = 500
_EROWS = _E // _ECHUNK
_EROWS_W = _EROWS // _NW

_mesh = plsc.VectorSubcoreMesh(
    core_axis_name="c", subcore_axis_name="s", num_cores=_NC, num_subcores=_NS
)


@functools.partial(
    pl.kernel,
    out_type=[
        jax.ShapeDtypeStruct((_NPAD,), jnp.float32),
        jax.ShapeDtypeStruct((_NPAD,), jnp.float32),
    ],
    mesh=_mesh,
    compiler_params=pltpu.CompilerParams(use_tc_tiling_on_sc=False),
    scratch_types=[
        pltpu.VMEM((_EROWS_W, _ECHUNK), jnp.int32),
        pltpu.VMEM((512,), jnp.float32),
        pltpu.VMEM((_NPS,), jnp.float32),
        pltpu.VMEM_SHARED((_NPAD,), jnp.float32),
    ],
)
def _sc_degree(edge_hbm, cnt0_hbm, cnt1_hbm, colv, ones_v, zbuf, acc):
    cid = lax.axis_index("c")
    sid = lax.axis_index("s")
    wid = sid * _NC + cid
    for k in range(32):
        ones_v[pl.ds(k * 16, 16)] = jnp.ones((16,), jnp.float32)

    def zfill(k, carry):
        zbuf[pl.ds(k * 16, 16)] = jnp.zeros((16,), jnp.float32)
        return carry

    lax.fori_loop(0, _NPS // 16, zfill, 0)
    pltpu.sync_copy(zbuf, acc.at[pl.ds(sid * _NPS, _NPS)])
    pltpu.sync_copy(edge_hbm.at[1, pl.ds(wid * _EROWS_W, _EROWS_W)], colv)
    plsc.subcore_barrier()

    def body(j, carry):
        pltpu.sync_copy(ones_v.at[pl.ds(0, _ECHUNK)], acc.at[colv.at[j]], add=True)
        return carry

    lax.fori_loop(0, _EROWS_W, body, 0)
    plsc.subcore_barrier()

    @pl.when(cid == 0)
    def _():
        pltpu.sync_copy(acc.at[pl.ds(sid * _NPS, _NPS)], cnt0_hbm.at[pl.ds(sid * _NPS, _NPS)])

    @pl.when(cid == 1)
    def _():
        pltpu.sync_copy(acc.at[pl.ds(sid * _NPS, _NPS)], cnt1_hbm.at[pl.ds(sid * _NPS, _NPS)])


def _make_sc_scatter(depth, nbuf, concat_out):
    if concat_out:
        out_type = [jax.ShapeDtypeStruct((_NPAD, 2 * depth), jnp.float32)]
    else:
        out_type = [
            jax.ShapeDtypeStruct((_NPAD, depth), jnp.float32),
            jax.ShapeDtypeStruct((_NPAD, depth), jnp.float32),
        ]

    @functools.partial(
        pl.kernel,
        out_type=out_type,
        mesh=_mesh,
        compiler_params=pltpu.CompilerParams(use_tc_tiling_on_sc=False),
        scratch_types=(
            [
                pltpu.VMEM((_EROWS_W, _ECHUNK), jnp.int32),
                pltpu.VMEM((_EROWS_W, _ECHUNK), jnp.int32),
            ]
            + [pltpu.VMEM((_ECHUNK, depth), jnp.float32)] * nbuf
            + [pltpu.VMEM_SHARED((_NPAD, depth), jnp.float32)]
            + [pltpu.SemaphoreType.DMA] * nbuf
        ),
    )
    def _sc_scatter(edge_hbm, y_hbm, zd_hbm, *rest):
        if concat_out:
            p01_hbm = rest[0]
            rest = rest[1:]
        else:
            p0_hbm, p1_hbm = rest[:2]
            rest = rest[2:]
        rowv, colv = rest[:2]
        bufs = rest[2:2 + nbuf]
        acc = rest[2 + nbuf]
        sems = rest[3 + nbuf:]
        cid = lax.axis_index("c")
        sid = lax.axis_index("s")
        wid = sid * _NC + cid

        @pl.when(cid == 0)
        def _():
            pltpu.sync_copy(y_hbm.at[pl.ds(sid * _NPS, _NPS)],
                            acc.at[pl.ds(sid * _NPS, _NPS)])

        @pl.when(cid == 1)
        def _():
            pltpu.sync_copy(zd_hbm.at[pl.ds(sid * _NPS, _NPS)],
                            acc.at[pl.ds(sid * _NPS, _NPS)])

        pltpu.sync_copy(edge_hbm.at[0, pl.ds(wid * _EROWS_W, _EROWS_W)], rowv)
        pltpu.sync_copy(edge_hbm.at[1, pl.ds(wid * _EROWS_W, _EROWS_W)], colv)
        plsc.subcore_barrier()

        for b in range(nbuf):
            pltpu.async_copy(y_hbm.at[rowv.at[b]], bufs[b], sems[b])

        def body(i, carry):
            for b in range(nbuf):
                j = nbuf * i + b
                pltpu.make_async_copy(y_hbm.at[rowv.at[j]], bufs[b], sems[b]).wait()
                pltpu.sync_copy(bufs[b], acc.at[colv.at[j]], add=True)

                @pl.when(j + nbuf < _EROWS_W)
                def _():
                    pltpu.async_copy(y_hbm.at[rowv.at[j + nbuf]], bufs[b], sems[b])

            return carry

        lax.fori_loop(0, _EROWS_W // nbuf, body, 0)
        plsc.subcore_barrier()

        if concat_out:
            pltpu.sync_copy(
                acc.at[pl.ds(sid * _NPS, _NPS)],
                p01_hbm.at[pl.ds(sid * _NPS, _NPS), pl.ds(cid * depth, depth)])
        else:
            @pl.when(cid == 0)
            def _():
                pltpu.sync_copy(acc.at[pl.ds(sid * _NPS, _NPS)],
                                p0_hbm.at[pl.ds(sid * _NPS, _NPS)])

            @pl.when(cid == 1)
            def _():
                pltpu.sync_copy(acc.at[pl.ds(sid * _NPS, _NPS)],
                                p1_hbm.at[pl.ds(sid * _NPS, _NPS)])

    return _sc_scatter


_sc_scatter_hid = _make_sc_scatter(_HID, 2, True)
_sc_scatter_out = _make_sc_scatter(_C, 4, True)


_R = 1000
_G = _N // _R


def _tc1a_body(x_ref, w1_ref, w2_ref, c_ref, y_ref, dinv_ref, o_ref):
    xw = jnp.dot(x_ref[...], w1_ref[...], preferred_element_type=jnp.float32)
    deg = c_ref[...] + 1.0
    dinv = lax.rsqrt(deg)
    y_ref[...] = xw * dinv
    dinv_ref[...] = dinv

    @pl.when(pl.program_id(0) == 0)
    def _():
        w1 = w1_ref[...]
        w2 = w2_ref[...]
        g1 = lax.dot_general(w1, w1, (((1,), (1,)), ((), ())),
                             preferred_element_type=jnp.float32)
        g2 = lax.dot_general(w2, w2, (((1,), (1,)), ((), ())),
                             preferred_element_type=jnp.float32)
        i1 = (lax.broadcasted_iota(jnp.int32, (_F_IN, _F_IN), 0)
              == lax.broadcasted_iota(jnp.int32, (_F_IN, _F_IN), 1)).astype(jnp.float32)
        i2 = (lax.broadcasted_iota(jnp.int32, (_HID, _HID), 0)
              == lax.broadcasted_iota(jnp.int32, (_HID, _HID), 1)).astype(jnp.float32)
        s1 = jnp.sum((g1 - i1) ** 2)
        s2 = jnp.sum((g2 - i2) ** 2)
        o_ref[...] = jnp.reshape(jnp.sqrt(s1) + jnp.sqrt(s2), (1, 1))


_tc1a = pl.pallas_call(
    _tc1a_body,
    grid=(_G,),
    in_specs=[
        pl.BlockSpec((_R, _F_IN), lambda i: (i, 0)),
        pl.BlockSpec((_F_IN, _HID), lambda i: (0, 0)),
        pl.BlockSpec((_HID, _C), lambda i: (0, 0)),
        pl.BlockSpec((_R, 1), lambda i: (i, 0)),
    ],
    out_specs=[
        pl.BlockSpec((_R, _HID), lambda i: (i, 0)),
        pl.BlockSpec((_R, 1), lambda i: (i, 0)),
        pl.BlockSpec((1, 1), lambda i: (0, 0)),
    ],
    out_shape=[
        jax.ShapeDtypeStruct((_NPAD, _HID), jnp.float32),
        jax.ShapeDtypeStruct((_N, 1), jnp.float32),
        jax.ShapeDtypeStruct((1, 1), jnp.float32),
    ],
)


def _tc2_body(p01_ref, dinv_ref, b1_ref, w2_ref, z_ref):
    dinv = dinv_ref[...]
    p01 = p01_ref[...]
    out1 = (p01[:, :_HID] + p01[:, _HID:]) * dinv + b1_ref[...]
    h = jnp.maximum(out1, 0.0)
    z_ref[...] = jnp.dot(h, w2_ref[...], preferred_element_type=jnp.float32) * dinv


_tc2 = pl.pallas_call(
    _tc2_body,
    grid=(_G,),
    in_specs=[
        pl.BlockSpec((_R, 2 * _HID), lambda i: (i, 0)),
        pl.BlockSpec((_R, 1), lambda i: (i, 0)),
        pl.BlockSpec((1, _HID), lambda i: (0, 0)),
        pl.BlockSpec((_HID, _C), lambda i: (0, 0)),
    ],
    out_specs=[pl.BlockSpec((_R, _C), lambda i: (i, 0))],
    out_shape=[jax.ShapeDtypeStruct((_NPAD, _C), jnp.float32)],
)


def _tc3_body(q01_ref, dinv_ref, b2_ref, logp_ref, xout_ref):
    q01 = q01_ref[...]
    xo = (q01[:, :_C] + q01[:, _C:]) * dinv_ref[...] + b2_ref[...]
    m = jnp.max(xo, axis=1, keepdims=True)
    t = xo - m
    lse = jnp.log(jnp.sum(jnp.exp(t), axis=1, keepdims=True))
    logp_ref[...] = t - lse
    xout_ref[...] = xo


_tc3 = pl.pallas_call(
    _tc3_body,
    grid=(_G,),
    in_specs=[
        pl.BlockSpec((_R, 2 * _C), lambda i: (i, 0)),
        pl.BlockSpec((_R, 1), lambda i: (i, 0)),
        pl.BlockSpec((1, _C), lambda i: (0, 0)),
    ],
    out_specs=[
        pl.BlockSpec((_R, _C), lambda i: (i, 0)),
        pl.BlockSpec((_R, _C), lambda i: (i, 0)),
    ],
    out_shape=[
        jax.ShapeDtypeStruct((_N, _C), jnp.float32),
        jax.ShapeDtypeStruct((_N, _C), jnp.float32),
    ],
)


def kernel(x, edge_index, W1, b1, W2, b2):
    edge_r = edge_index.reshape(2, _EROWS, _ECHUNK)
    z64 = jnp.zeros((_NPAD, _HID), jnp.float32)
    z16 = jnp.zeros((_NPAD, _C), jnp.float32)

    cnt0, cnt1 = _sc_degree(edge_r)
    cnt = (cnt0 + cnt1).reshape(_NPAD, 1)[: _N]
    y1, dinv, orto = _tc1a(x, W1, W2, cnt)
    (p01,) = _sc_scatter_hid(edge_r, y1, z64)
    (z2,) = _tc2(p01, dinv, b1.reshape(1, _HID), W2)
    (q01,) = _sc_scatter_out(edge_r, z2, z16)
    logp, xout = _tc3(q01, dinv, b2.reshape(1, _C))
    return (logp, xout, orto.reshape(()))


# speedup vs baseline: 1.0843x; 1.0327x over previous
import functools

import jax
import jax.numpy as jnp
from jax import lax
from jax.experimental import pallas as pl
from jax.experimental.pallas import tpu as pltpu
from jax.experimental.pallas import tpu_sc as plsc

_N = 10000
_E = 320000
_F_IN = 128
_HID = 64
_C = 16

_NC = 2
_NS = 16
_NW = _NC * _NS
_CHUNK = 125
_ROWS = _E // _CHUNK
_ROWS_W = _ROWS // _NW
_NPAD = 10240
_NPS = _NPAD // _NS

_ECHUNK = 250
_EROWS = _E // _ECHUNK
_EROWS_W = _EROWS // _NW

_mesh = plsc.VectorSubcoreMesh(
    core_axis_name="c", subcore_axis_name="s", num_cores=_NC, num_subcores=_NS
)


@functools.partial(
    pl.kernel,
    out_type=[
        jax.ShapeDtypeStruct((_NPAD,), jnp.float32),
        jax.ShapeDtypeStruct((_NPAD,), jnp.float32),
    ],
    mesh=_mesh,
    compiler_params=pltpu.CompilerParams(use_tc_tiling_on_sc=False),
    scratch_types=[
        pltpu.VMEM((_EROWS_W, _ECHUNK), jnp.int32),
        pltpu.VMEM((512,), jnp.float32),
        pltpu.VMEM((_NPS,), jnp.float32),
        pltpu.VMEM_SHARED((_NPAD,), jnp.float32),
    ],
)
def _sc_degree(edge_hbm, cnt0_hbm, cnt1_hbm, colv, ones_v, zbuf, acc):
    cid = lax.axis_index("c")
    sid = lax.axis_index("s")
    wid = sid * _NC + cid
    for k in range(32):
        ones_v[pl.ds(k * 16, 16)] = jnp.ones((16,), jnp.float32)

    def zfill(k, carry):
        zbuf[pl.ds(k * 16, 16)] = jnp.zeros((16,), jnp.float32)
        return carry

    lax.fori_loop(0, _NPS // 16, zfill, 0)
    pltpu.sync_copy(zbuf, acc.at[pl.ds(sid * _NPS, _NPS)])
    pltpu.sync_copy(edge_hbm.at[1, pl.ds(wid * _EROWS_W, _EROWS_W)], colv)
    plsc.subcore_barrier()

    def body(j, carry):
        pltpu.sync_copy(ones_v.at[pl.ds(0, _ECHUNK)], acc.at[colv.at[j]], add=True)
        return carry

    lax.fori_loop(0, _EROWS_W, body, 0)
    plsc.subcore_barrier()

    @pl.when(cid == 0)
    def _():
        pltpu.sync_copy(acc.at[pl.ds(sid * _NPS, _NPS)], cnt0_hbm.at[pl.ds(sid * _NPS, _NPS)])

    @pl.when(cid == 1)
    def _():
        pltpu.sync_copy(acc.at[pl.ds(sid * _NPS, _NPS)], cnt1_hbm.at[pl.ds(sid * _NPS, _NPS)])


def _make_sc_scatter(depth, nbuf, concat_out):
    if concat_out:
        out_type = [jax.ShapeDtypeStruct((_NPAD, 2 * depth), jnp.float32)]
    else:
        out_type = [
            jax.ShapeDtypeStruct((_NPAD, depth), jnp.float32),
            jax.ShapeDtypeStruct((_NPAD, depth), jnp.float32),
        ]

    @functools.partial(
        pl.kernel,
        out_type=out_type,
        mesh=_mesh,
        compiler_params=pltpu.CompilerParams(use_tc_tiling_on_sc=False),
        scratch_types=(
            [
                pltpu.VMEM((_EROWS_W, _ECHUNK), jnp.int32),
                pltpu.VMEM((_EROWS_W, _ECHUNK), jnp.int32),
            ]
            + [pltpu.VMEM((_ECHUNK, depth), jnp.float32)] * nbuf
            + [pltpu.VMEM_SHARED((_NPAD, depth), jnp.float32)]
            + [pltpu.SemaphoreType.DMA] * nbuf
        ),
    )
    def _sc_scatter(edge_hbm, y_hbm, zd_hbm, *rest):
        if concat_out:
            p01_hbm = rest[0]
            rest = rest[1:]
        else:
            p0_hbm, p1_hbm = rest[:2]
            rest = rest[2:]
        rowv, colv = rest[:2]
        bufs = rest[2:2 + nbuf]
        acc = rest[2 + nbuf]
        sems = rest[3 + nbuf:]
        cid = lax.axis_index("c")
        sid = lax.axis_index("s")
        wid = sid * _NC + cid

        @pl.when(cid == 0)
        def _():
            pltpu.sync_copy(y_hbm.at[pl.ds(sid * _NPS, _NPS)],
                            acc.at[pl.ds(sid * _NPS, _NPS)])

        @pl.when(cid == 1)
        def _():
            pltpu.sync_copy(zd_hbm.at[pl.ds(sid * _NPS, _NPS)],
                            acc.at[pl.ds(sid * _NPS, _NPS)])

        pltpu.sync_copy(edge_hbm.at[0, pl.ds(wid * _EROWS_W, _EROWS_W)], rowv)
        pltpu.sync_copy(edge_hbm.at[1, pl.ds(wid * _EROWS_W, _EROWS_W)], colv)
        plsc.subcore_barrier()

        for b in range(nbuf):
            pltpu.async_copy(y_hbm.at[rowv.at[b]], bufs[b], sems[b])

        def body(i, carry):
            for b in range(nbuf):
                j = nbuf * i + b
                pltpu.make_async_copy(y_hbm.at[rowv.at[j]], bufs[b], sems[b]).wait()
                pltpu.sync_copy(bufs[b], acc.at[colv.at[j]], add=True)

                @pl.when(j + nbuf < _EROWS_W)
                def _():
                    pltpu.async_copy(y_hbm.at[rowv.at[j + nbuf]], bufs[b], sems[b])

            return carry

        lax.fori_loop(0, _EROWS_W // nbuf, body, 0)
        plsc.subcore_barrier()

        if concat_out:
            pltpu.sync_copy(
                acc.at[pl.ds(sid * _NPS, _NPS)],
                p01_hbm.at[pl.ds(sid * _NPS, _NPS), pl.ds(cid * depth, depth)])
        else:
            @pl.when(cid == 0)
            def _():
                pltpu.sync_copy(acc.at[pl.ds(sid * _NPS, _NPS)],
                                p0_hbm.at[pl.ds(sid * _NPS, _NPS)])

            @pl.when(cid == 1)
            def _():
                pltpu.sync_copy(acc.at[pl.ds(sid * _NPS, _NPS)],
                                p1_hbm.at[pl.ds(sid * _NPS, _NPS)])

    return _sc_scatter


_sc_scatter_hid = _make_sc_scatter(_HID, 4, True)
_sc_scatter_out = _make_sc_scatter(_C, 4, True)


_R = 1000
_G = _N // _R


def _tc1a_body(x_ref, w1_ref, w2_ref, c_ref, y_ref, dinv_ref, o_ref):
    xw = jnp.dot(x_ref[...], w1_ref[...], preferred_element_type=jnp.float32)
    deg = c_ref[...] + 1.0
    dinv = lax.rsqrt(deg)
    y_ref[...] = xw * dinv
    dinv_ref[...] = dinv

    @pl.when(pl.program_id(0) == 0)
    def _():
        w1 = w1_ref[...]
        w2 = w2_ref[...]
        g1 = lax.dot_general(w1, w1, (((1,), (1,)), ((), ())),
                             preferred_element_type=jnp.float32)
        g2 = lax.dot_general(w2, w2, (((1,), (1,)), ((), ())),
                             preferred_element_type=jnp.float32)
        i1 = (lax.broadcasted_iota(jnp.int32, (_F_IN, _F_IN), 0)
              == lax.broadcasted_iota(jnp.int32, (_F_IN, _F_IN), 1)).astype(jnp.float32)
        i2 = (lax.broadcasted_iota(jnp.int32, (_HID, _HID), 0)
              == lax.broadcasted_iota(jnp.int32, (_HID, _HID), 1)).astype(jnp.float32)
        s1 = jnp.sum((g1 - i1) ** 2)
        s2 = jnp.sum((g2 - i2) ** 2)
        o_ref[...] = jnp.reshape(jnp.sqrt(s1) + jnp.sqrt(s2), (1, 1))


_tc1a = pl.pallas_call(
    _tc1a_body,
    grid=(_G,),
    in_specs=[
        pl.BlockSpec((_R, _F_IN), lambda i: (i, 0)),
        pl.BlockSpec((_F_IN, _HID), lambda i: (0, 0)),
        pl.BlockSpec((_HID, _C), lambda i: (0, 0)),
        pl.BlockSpec((_R, 1), lambda i: (i, 0)),
    ],
    out_specs=[
        pl.BlockSpec((_R, _HID), lambda i: (i, 0)),
        pl.BlockSpec((_R, 1), lambda i: (i, 0)),
        pl.BlockSpec((1, 1), lambda i: (0, 0)),
    ],
    out_shape=[
        jax.ShapeDtypeStruct((_NPAD, _HID), jnp.float32),
        jax.ShapeDtypeStruct((_N, 1), jnp.float32),
        jax.ShapeDtypeStruct((1, 1), jnp.float32),
    ],
)


def _tc2_body(p01_ref, dinv_ref, b1_ref, w2_ref, z_ref):
    dinv = dinv_ref[...]
    p01 = p01_ref[...]
    out1 = (p01[:, :_HID] + p01[:, _HID:]) * dinv + b1_ref[...]
    h = jnp.maximum(out1, 0.0)
    z_ref[...] = jnp.dot(h, w2_ref[...], preferred_element_type=jnp.float32) * dinv


_tc2 = pl.pallas_call(
    _tc2_body,
    grid=(_G,),
    in_specs=[
        pl.BlockSpec((_R, 2 * _HID), lambda i: (i, 0)),
        pl.BlockSpec((_R, 1), lambda i: (i, 0)),
        pl.BlockSpec((1, _HID), lambda i: (0, 0)),
        pl.BlockSpec((_HID, _C), lambda i: (0, 0)),
    ],
    out_specs=[pl.BlockSpec((_R, _C), lambda i: (i, 0))],
    out_shape=[jax.ShapeDtypeStruct((_NPAD, _C), jnp.float32)],
)


def _tc3_body(q01_ref, dinv_ref, b2_ref, logp_ref, xout_ref):
    q01 = q01_ref[...]
    xo = (q01[:, :_C] + q01[:, _C:]) * dinv_ref[...] + b2_ref[...]
    m = jnp.max(xo, axis=1, keepdims=True)
    t = xo - m
    lse = jnp.log(jnp.sum(jnp.exp(t), axis=1, keepdims=True))
    logp_ref[...] = t - lse
    xout_ref[...] = xo


_tc3 = pl.pallas_call(
    _tc3_body,
    grid=(_G,),
    in_specs=[
        pl.BlockSpec((_R, 2 * _C), lambda i: (i, 0)),
        pl.BlockSpec((_R, 1), lambda i: (i, 0)),
        pl.BlockSpec((1, _C), lambda i: (0, 0)),
    ],
    out_specs=[
        pl.BlockSpec((_R, _C), lambda i: (i, 0)),
        pl.BlockSpec((_R, _C), lambda i: (i, 0)),
    ],
    out_shape=[
        jax.ShapeDtypeStruct((_N, _C), jnp.float32),
        jax.ShapeDtypeStruct((_N, _C), jnp.float32),
    ],
)


def kernel(x, edge_index, W1, b1, W2, b2):
    edge_r = edge_index.reshape(2, _EROWS, _ECHUNK)
    z64 = jnp.zeros((_NPAD, _HID), jnp.float32)
    z16 = jnp.zeros((_NPAD, _C), jnp.float32)

    cnt0, cnt1 = _sc_degree(edge_r)
    cnt = (cnt0 + cnt1).reshape(_NPAD, 1)[: _N]
    y1, dinv, orto = _tc1a(x, W1, W2, cnt)
    (p01,) = _sc_scatter_hid(edge_r, y1, z64)
    (z2,) = _tc2(p01, dinv, b1.reshape(1, _HID), W2)
    (q01,) = _sc_scatter_out(edge_r, z2, z16)
    logp, xout = _tc3(q01, dinv, b2.reshape(1, _C))
    return (logp, xout, orto.reshape(()))
